# baseline (speedup 1.0000x reference)
import functools
import jax
import jax.numpy as jnp
from jax import lax
from jax.experimental import pallas as pl
from jax.experimental.pallas import tpu as pltpu
from jax.experimental.pallas import tpu_sc as plsc

_LANE_OFFS = (0, 16, 32, 34)


def _mm_body(emb_ref, att_ref, w_ref, a_ref, b_ref):
    BL = emb_ref.shape[0] * emb_ref.shape[1]
    D = emb_ref.shape[2]
    e2d = emb_ref[...].reshape(BL, D)
    neg_inf = jnp.float32(-jnp.inf)
    a = jax.lax.dot_general(e2d, w_ref[:, :D], (((1,), (1,)), ((), ())),
                            preferred_element_type=jnp.float32)
    bv = jax.lax.dot_general(e2d, w_ref[:, D:], (((1,), (1,)), ((), ())),
                             preferred_element_type=jnp.float32)
    a_ref[...] = jnp.where(att_ref[...] > 0, a, neg_inf)
    b_ref[...] = jnp.where(att_ref[...] > 0, bv, neg_inf)


def _sc_body(a_hbm, bv_hbm, out_hbm, a_vm, b_vm, ob):
    L = 64
    JW = 16
    cid = lax.axis_index("c")
    sid = lax.axis_index("s")
    w = sid * 2 + cid
    b = w // 4
    j0 = (w % 4) * JW

    pltpu.sync_copy(a_hbm.at[pl.ds(b * L, L), :], a_vm)
    pltpu.sync_copy(bv_hbm.at[pl.ds(b * L + j0, JW), :], b_vm)

    neg_inf = jnp.full((16,), -jnp.inf, dtype=jnp.float32)

    def step(jj, carry):
        j = j0 + jj
        brow = [b_vm[jj, pl.ds(off, 16)] for off in _LANE_OFFS]
        for k in range(L):
            for t, off in enumerate(_LANE_OFFS):
                ob[k, pl.ds(off, 16)] = a_vm[k, pl.ds(off, 16)] + brow[t]
        for off in _LANE_OFFS:
            ob[j, pl.ds(off, 16)] = neg_inf
        pltpu.sync_copy(ob, out_hbm.at[b, pl.ds(j * L, L), :])
        return carry

    lax.fori_loop(0, JW, step, 0)


def kernel(emb_sentences, att_sentences, W):
    B, L, D = emb_sentences.shape
    NL = W.shape[0]
    att_col = att_sentences.astype(jnp.float32).reshape(B * L, 1)

    a_part, b_part = pl.pallas_call(
        _mm_body,
        in_specs=[
            pl.BlockSpec((B, L, D), lambda: (0, 0, 0)),
            pl.BlockSpec((B * L, 1), lambda: (0, 0)),
            pl.BlockSpec((NL, 2 * D), lambda: (0, 0)),
        ],
        out_specs=[
            pl.BlockSpec((B * L, NL), lambda: (0, 0)),
            pl.BlockSpec((B * L, NL), lambda: (0, 0)),
        ],
        out_shape=[
            jax.ShapeDtypeStruct((B * L, NL), jnp.float32),
            jax.ShapeDtypeStruct((B * L, NL), jnp.float32),
        ],
    )(emb_sentences, att_col, W)

    sc = functools.partial(
        pl.kernel,
        out_type=jax.ShapeDtypeStruct((B, L * L, NL), jnp.float32),
        mesh=plsc.VectorSubcoreMesh(core_axis_name="c", subcore_axis_name="s"),
        scratch_types=[
            pltpu.VMEM((L, NL), jnp.float32),
            pltpu.VMEM((16, NL), jnp.float32),
            pltpu.VMEM((L, NL), jnp.float32),
        ],
        compiler_params=pltpu.CompilerParams(use_tc_tiling_on_sc=True),
    )(_sc_body)
    return sc(a_part, b_part)


# speedup vs baseline: 3.8059x; 3.8059x over previous
import jax
import jax.numpy as jnp
import numpy as np
from jax.experimental import pallas as pl
from jax.experimental.pallas import tpu as pltpu

_PC = 8


def _body(emb_ref, att_ref, w_ref, tkj_ref, diag_ref, perm_ref, rep_ref,
          out_ref, mnb_ref, mn_ref):
    NL, D2 = w_ref.shape
    D = D2 // 2
    B, L, _ = emb_ref.shape
    pc = pl.program_id(0)
    neg_inf = jnp.float32(-jnp.inf)

    @pl.when(pc == 0)
    def _():
        for b in range(B):
            e_b = emb_ref[b]
            a_b = jax.lax.dot_general(
                w_ref[:, :D], e_b, (((1,), (1,)), ((), ())),
                preferred_element_type=jnp.float32)
            b_b = jax.lax.dot_general(
                w_ref[:, D:], e_b, (((1,), (1,)), ((), ())),
                preferred_element_type=jnp.float32)
            mnb_ref[b * NL:(b + 1) * NL, :L] = a_b
            mnb_ref[b * NL:(b + 1) * NL, L:] = b_b
        mn_ref[...] = jax.lax.dot_general(
            perm_ref[...], mnb_ref[...], (((1,), (0,)), ((), ())),
            preferred_element_type=jnp.float32)

    cw = tkj_ref.shape[1]
    planes = jax.lax.dot_general(mn_ref[...], tkj_ref[...],
                                 (((1,), (0,)), ((), ())),
                                 preferred_element_type=jnp.float32)
    att = att_ref[...]
    att_kj = jax.lax.dot_general(att, tkj_ref[:L] + tkj_ref[L:],
                                 (((1,), (0,)), ((), ())),
                                 preferred_element_type=jnp.float32)
    badf = jnp.where((att_kj < 2.0) | (diag_ref[...] > 0), 1.0, 0.0)
    bad400 = jax.lax.dot_general(rep_ref[...], badf, (((1,), (0,)), ((), ())),
                                 preferred_element_type=jnp.float32)
    out_ref[...] = jnp.where(bad400 > 0, neg_inf, planes)


def kernel(emb_sentences, att_sentences, W):
    B, L, D = emb_sentences.shape
    NL = W.shape[0]
    LL = L * L
    CW = LL // _PC
    att_f = att_sentences.astype(jnp.float32)

    p = np.arange(LL)
    tile_k = p % L == np.arange(L)[:, None]
    tile_j = p // L == np.arange(L)[:, None]
    tkj = jnp.asarray(np.concatenate([tile_k, tile_j], 0), dtype=jnp.float32)
    diag = jnp.asarray((p // L == p % L)[None, :], dtype=jnp.float32)
    perm_np = np.zeros((NL * B, NL * B), dtype=np.float32)
    lidx = np.arange(NL * B)
    perm_np[lidx, (lidx % B) * NL + lidx // B] = 1.0
    perm = jnp.asarray(perm_np)
    rep = jnp.asarray(
        np.arange(B)[None, :] == (np.arange(NL * B) % B)[:, None],
        dtype=jnp.float32)

    out2d = pl.pallas_call(
        _body,
        grid=(_PC,),
        in_specs=[
            pl.BlockSpec((B, L, D), lambda pc: (0, 0, 0)),
            pl.BlockSpec((B, L), lambda pc: (0, 0)),
            pl.BlockSpec((NL, 2 * D), lambda pc: (0, 0)),
            pl.BlockSpec((2 * L, CW), lambda pc: (0, pc)),
            pl.BlockSpec((1, CW), lambda pc: (0, pc)),
            pl.BlockSpec((NL * B, NL * B), lambda pc: (0, 0)),
            pl.BlockSpec((NL * B, B), lambda pc: (0, 0)),
        ],
        out_specs=pl.BlockSpec((NL * B, CW), lambda pc: (0, pc)),
        out_shape=jax.ShapeDtypeStruct((NL * B, LL), jnp.float32),
        scratch_shapes=[
            pltpu.VMEM((NL * B, 2 * L), jnp.float32),
            pltpu.VMEM((NL * B, 2 * L), jnp.float32),
        ],
    )(emb_sentences, att_f, W, tkj, diag, perm, rep)
    return jnp.transpose(out2d.reshape(NL, B, LL), (1, 2, 0))
